# baseline (speedup 1.0000x reference)
import functools

import jax
import jax.numpy as jnp
from jax import lax
from jax.experimental import pallas as pl
from jax.experimental.pallas import tpu as pltpu
from jax.experimental.pallas import tpu_sc as plsc

_B = 4096
_L = 200
_TD = 64
_NC = 2
_NS = 16
_NW = _NC * _NS
_BPW = _B // _NW
_CHA = 96
_CHB = 104
_NBUF = 4


_FC = 2048


def _format_body(tT_ref, out_ref):
  x = tT_ref[...]
  n = jnp.sqrt(jnp.sum(x * x, axis=0, keepdims=True))
  s = jnp.where(n > 1.0, 1.0 / jnp.maximum(n, 1e-12), 1.0)
  eye = (lax.broadcasted_iota(jnp.int32, (64, 64), 0)
         == lax.broadcasted_iota(jnp.int32, (64, 64), 1)).astype(jnp.float32)
  y = lax.dot_general(x * s, eye, (((0,), (0,)), ((), ())),
                      precision=lax.Precision.HIGHEST,
                      preferred_element_type=jnp.float32)
  out_ref[:, 0:64] = y[0:_FC // 2]
  out_ref[:, 64:128] = y[_FC // 2:]


def _format_table(tT, interpret=False):
  grid = (1000000 + _FC - 1) // _FC
  return pl.pallas_call(
      _format_body,
      grid=(grid,),
      in_specs=[pl.BlockSpec((64, _FC), lambda i: (0, i))],
      out_specs=pl.BlockSpec((_FC // 2, 128), lambda i: (i, 0)),
      out_shape=jax.ShapeDtypeStruct((500000, 128), jnp.float32),
      interpret=interpret,
  )(tT)


def _accum_chunk(rows, acc):
  def row_body(r, acc):
    a0, a1, a2, a3 = acc
    return (a0 + rows[r, pl.ds(0, 16)],
            a1 + rows[r, pl.ds(16, 16)],
            a2 + rows[r, pl.ds(32, 16)],
            a3 + rows[r, pl.ds(48, 16)])

  n = rows.shape[0]
  return lax.fori_loop(0, n, row_body, acc, unroll=4)


def _make_text_bag(interpret=False):
  mesh = plsc.VectorSubcoreMesh(
      core_axis_name="c", subcore_axis_name="s",
      num_cores=_NC, num_subcores=_NS)

  @functools.partial(
      pl.kernel,
      out_type=jax.ShapeDtypeStruct((_B, _TD), jnp.float32),
      mesh=mesh,
      scratch_types=[
          pltpu.VMEM((_BPW, _L), jnp.int32),
          pltpu.VMEM((_NBUF, _CHA, _TD), jnp.float32),
          pltpu.VMEM((_NBUF, _CHB, _TD), jnp.float32),
          pltpu.VMEM((_BPW, _TD), jnp.float32),
      ] + [pltpu.SemaphoreType.DMA] * (2 * _NBUF),
      compiler_params=pltpu.CompilerParams(use_tc_tiling_on_sc=False),
      interpret=interpret,
  )
  def text_bag(tbl, idx, out, idx_v, rab, rbb, obuf, *sems):
    wid = lax.axis_index("s") * _NC + lax.axis_index("c")

    def cha(d):
      return idx_v.at[d, pl.ds(0, _CHA)]

    def chb(d):
      return idx_v.at[d, pl.ds(_CHA, _CHB)]

    pltpu.sync_copy(idx.at[pl.ds(wid * _BPW, _BPW)], idx_v)
    for b in range(_NBUF):
      pltpu.async_copy(tbl.at[cha(b)], rab.at[b], sems[2 * b])
      pltpu.async_copy(tbl.at[chb(b)], rbb.at[b], sems[2 * b + 1])

    def store(d, acc):
      a0, a1, a2, a3 = acc
      obuf[d, pl.ds(0, 16)] = a0
      obuf[d, pl.ds(16, 16)] = a1
      obuf[d, pl.ds(32, 16)] = a2
      obuf[d, pl.ds(48, 16)] = a3

    def one_doc(d, b):
      zero = jnp.zeros((16,), jnp.float32)
      sa, sb = sems[2 * b], sems[2 * b + 1]
      pltpu.make_async_copy(tbl.at[cha(d)], rab.at[b], sa).wait()
      acc = _accum_chunk(rab.at[b], (zero, zero, zero, zero))

      @pl.when(d + _NBUF < _BPW)
      def _():
        pltpu.async_copy(tbl.at[cha(d + _NBUF)], rab.at[b], sa)

      pltpu.make_async_copy(tbl.at[chb(d)], rbb.at[b], sb).wait()
      acc = _accum_chunk(rbb.at[b], acc)

      @pl.when(d + _NBUF < _BPW)
      def _():
        pltpu.async_copy(tbl.at[chb(d + _NBUF)], rbb.at[b], sb)

      store(d, acc)

    def group_body(g, carry):
      for b in range(_NBUF):
        one_doc(_NBUF * g + b, b)
      return carry

    lax.fori_loop(0, _BPW // _NBUF, group_body, 0)
    pltpu.sync_copy(obuf, out.at[pl.ds(wid * _BPW, _BPW)])

  return text_bag


_text_bag_cache = {}


def _text_bag(tbl, idx3):
  if "k" not in _text_bag_cache:
    _text_bag_cache["k"] = _make_text_bag()
  return _text_bag_cache["k"](tbl, idx3)


_BLK = 512


def _head_body(lang_ref, brk_ref, caps_ref, tsum_ref, lt_ref, bt_ref,
               ct_ref, tw_ref, tb_ref, pa_ref, dw_ref, db_ref, out_ref):
  f32 = jnp.float32

  def renorm(t):
    n = jnp.sqrt(jnp.sum(t * t, axis=1, keepdims=True))
    return t * jnp.where(n > 1.0, 1.0 / jnp.maximum(n, 1e-12), 1.0)

  rb = renorm(bt_ref[...])
  rc = renorm(ct_ref[...])
  rl = renorm(lt_ref[...])

  brk = brk_ref[...]
  caps = caps_ref[...]
  bsum = jnp.zeros((_BLK, 8), f32)
  csum = jnp.zeros((_BLK, 8), f32)
  for k in range(4):
    cb = jnp.sum((brk == k).astype(f32), axis=1, keepdims=True)
    cc = jnp.sum((caps == k).astype(f32), axis=1, keepdims=True)
    bsum = bsum + cb * rb[k:k + 1, :]
    csum = csum + cc * rc[k:k + 1, :]

  tw = tw_ref[...]
  dnT = (((1,), (1,)), ((), ()))
  summed = (
      lax.dot_general(tsum_ref[...], tw[:, 16:], dnT, preferred_element_type=f32)
      + lax.dot_general(bsum, tw[:, 0:8], dnT, preferred_element_type=f32)
      + lax.dot_general(csum, tw[:, 8:16], dnT, preferred_element_type=f32)
      + jnp.float32(_L) * tb_ref[...])
  a = pa_ref[0, 0]
  phrase = jnp.where(summed >= 0, summed, a * summed)

  oh = (lax.broadcasted_iota(jnp.int32, (_BLK, 128), 1) == lang_ref[...]).astype(f32)
  dn = (((1,), (0,)), ((), ()))
  lemb = lax.dot_general(oh, rl, dn, preferred_element_type=f32)

  dw = dw_ref[...]
  out_ref[...] = (
      lax.dot_general(lemb, dw[:, 0:16], dnT, preferred_element_type=f32)
      + lax.dot_general(phrase, dw[:, 16:], dnT, preferred_element_type=f32)
      + db_ref[...])


def _head(lang2, brk, caps, tsum, lt, bt, ct, tw, tb2, pa2, dw, db2,
          interpret=False):
  def ix(i):
    return (i, 0)

  def fx(i):
    return (0, 0)

  return pl.pallas_call(
      _head_body,
      grid=(_B // _BLK,),
      in_specs=[
          pl.BlockSpec((_BLK, 1), ix),
          pl.BlockSpec((_BLK, _L), ix),
          pl.BlockSpec((_BLK, _L), ix),
          pl.BlockSpec((_BLK, _TD), ix),
          pl.BlockSpec((128, 16), fx),
          pl.BlockSpec((4, 8), fx),
          pl.BlockSpec((4, 8), fx),
          pl.BlockSpec((64, 80), fx),
          pl.BlockSpec((1, 64), fx),
          pl.BlockSpec(memory_space=pltpu.SMEM),
          pl.BlockSpec((128, 80), fx),
          pl.BlockSpec((1, 128), fx),
      ],
      out_specs=pl.BlockSpec((_BLK, 128), ix),
      out_shape=jax.ShapeDtypeStruct((_B, 128), jnp.float32),
      interpret=interpret,
  )(lang2, brk, caps, tsum, lt, bt, ct, tw, tb2, pa2, dw, db2)


def kernel(lang_inputs, toks_inputs, lang_table, break_table, caps_table,
           text_table, toks_W, toks_b, prelu_a, docs_W, docs_b):
  brk = toks_inputs[0]
  caps = toks_inputs[1]
  tidx = toks_inputs[2].astype(jnp.int32)
  fmt = _format_table(text_table.T)
  b = tidx & 2047
  sidx = (tidx - b) + ((b & 1023) << 1) + (b >> 10)
  tsum = _text_bag(fmt.reshape(1000000, 64), sidx)

  pa2 = jnp.reshape(prelu_a, (1, 1))
  tb2 = jnp.reshape(toks_b, (1, 64))
  db2 = jnp.reshape(docs_b, (1, 128))
  return _head(lang_inputs, brk, caps, tsum, lang_table, break_table,
               caps_table, toks_W, tb2, pa2, docs_W, db2)


# speedup vs baseline: 1.9147x; 1.9147x over previous
import functools

import jax
import jax.numpy as jnp
from jax import lax
from jax.experimental import pallas as pl
from jax.experimental.pallas import tpu as pltpu
from jax.experimental.pallas import tpu_sc as plsc

_B = 4096
_L = 200
_TD = 64
_NC = 2
_NS = 16
_NW = _NC * _NS
_BPW = _B // _NW
_CHA = 96
_CHB = 104
_NBUF = 4


_FC = 8192


def _format_body(tT_ref, out_ref):
  x = tT_ref[...]
  n = jnp.sqrt(jnp.sum(x * x, axis=0, keepdims=True))
  s = jnp.where(n > 1.0, 1.0 / jnp.maximum(n, 1e-12), 1.0)
  y = jnp.swapaxes(x * s, 0, 1)
  out_ref[:, 0:64] = y[0:_FC // 2]
  out_ref[:, 64:128] = y[_FC // 2:]


def _format_table(tT, interpret=False):
  grid = (1000000 + _FC - 1) // _FC
  return pl.pallas_call(
      _format_body,
      grid=(grid,),
      in_specs=[pl.BlockSpec((64, _FC), lambda i: (0, i))],
      out_specs=pl.BlockSpec((_FC // 2, 128), lambda i: (i, 0)),
      out_shape=jax.ShapeDtypeStruct((500000, 128), jnp.float32),
      interpret=interpret,
  )(tT)


def _accum_chunk(rows, acc):
  def row_body(r, acc):
    a0, a1, a2, a3 = acc
    return (a0 + rows[r, pl.ds(0, 16)],
            a1 + rows[r, pl.ds(16, 16)],
            a2 + rows[r, pl.ds(32, 16)],
            a3 + rows[r, pl.ds(48, 16)])

  n = rows.shape[0]
  return lax.fori_loop(0, n, row_body, acc, unroll=4)


def _make_text_bag(interpret=False):
  mesh = plsc.VectorSubcoreMesh(
      core_axis_name="c", subcore_axis_name="s",
      num_cores=_NC, num_subcores=_NS)

  @functools.partial(
      pl.kernel,
      out_type=jax.ShapeDtypeStruct((_B, _TD), jnp.float32),
      mesh=mesh,
      scratch_types=[
          pltpu.VMEM((_BPW, _L), jnp.int32),
          pltpu.VMEM((_NBUF, _CHA, _TD), jnp.float32),
          pltpu.VMEM((_NBUF, _CHB, _TD), jnp.float32),
          pltpu.VMEM((_BPW, _TD), jnp.float32),
      ] + [pltpu.SemaphoreType.DMA] * (2 * _NBUF),
      compiler_params=pltpu.CompilerParams(use_tc_tiling_on_sc=False),
      interpret=interpret,
  )
  def text_bag(tbl, idx, out, idx_v, rab, rbb, obuf, *sems):
    wid = lax.axis_index("s") * _NC + lax.axis_index("c")

    def cha(d):
      return idx_v.at[d, pl.ds(0, _CHA)]

    def chb(d):
      return idx_v.at[d, pl.ds(_CHA, _CHB)]

    pltpu.sync_copy(idx.at[pl.ds(wid * _BPW, _BPW)], idx_v)
    for b in range(_NBUF):
      pltpu.async_copy(tbl.at[cha(b)], rab.at[b], sems[2 * b])
      pltpu.async_copy(tbl.at[chb(b)], rbb.at[b], sems[2 * b + 1])

    def store(d, acc):
      a0, a1, a2, a3 = acc
      obuf[d, pl.ds(0, 16)] = a0
      obuf[d, pl.ds(16, 16)] = a1
      obuf[d, pl.ds(32, 16)] = a2
      obuf[d, pl.ds(48, 16)] = a3

    def one_doc(d, b):
      zero = jnp.zeros((16,), jnp.float32)
      sa, sb = sems[2 * b], sems[2 * b + 1]
      pltpu.make_async_copy(tbl.at[cha(d)], rab.at[b], sa).wait()
      acc = _accum_chunk(rab.at[b], (zero, zero, zero, zero))

      @pl.when(d + _NBUF < _BPW)
      def _():
        pltpu.async_copy(tbl.at[cha(d + _NBUF)], rab.at[b], sa)

      pltpu.make_async_copy(tbl.at[chb(d)], rbb.at[b], sb).wait()
      acc = _accum_chunk(rbb.at[b], acc)

      @pl.when(d + _NBUF < _BPW)
      def _():
        pltpu.async_copy(tbl.at[chb(d + _NBUF)], rbb.at[b], sb)

      store(d, acc)

    def group_body(g, carry):
      for b in range(_NBUF):
        one_doc(_NBUF * g + b, b)
      return carry

    lax.fori_loop(0, _BPW // _NBUF, group_body, 0)
    pltpu.sync_copy(obuf, out.at[pl.ds(wid * _BPW, _BPW)])

  return text_bag


_text_bag_cache = {}


def _text_bag(tbl, idx3):
  if "k" not in _text_bag_cache:
    _text_bag_cache["k"] = _make_text_bag()
  return _text_bag_cache["k"](tbl, idx3)


_BLK = 512


def _head_body(lang_ref, brk_ref, caps_ref, tsum_ref, lt_ref, bt_ref,
               ct_ref, tw_ref, tb_ref, pa_ref, dw_ref, db_ref, out_ref):
  f32 = jnp.float32

  def renorm(t):
    n = jnp.sqrt(jnp.sum(t * t, axis=1, keepdims=True))
    return t * jnp.where(n > 1.0, 1.0 / jnp.maximum(n, 1e-12), 1.0)

  rb = renorm(bt_ref[...])
  rc = renorm(ct_ref[...])
  rl = renorm(lt_ref[...])

  brk = brk_ref[...]
  caps = caps_ref[...]
  bsum = jnp.zeros((_BLK, 8), f32)
  csum = jnp.zeros((_BLK, 8), f32)
  for k in range(4):
    cb = jnp.sum((brk == k).astype(f32), axis=1, keepdims=True)
    cc = jnp.sum((caps == k).astype(f32), axis=1, keepdims=True)
    bsum = bsum + cb * rb[k:k + 1, :]
    csum = csum + cc * rc[k:k + 1, :]

  tw = tw_ref[...]
  dnT = (((1,), (1,)), ((), ()))
  summed = (
      lax.dot_general(tsum_ref[...], tw[:, 16:], dnT, preferred_element_type=f32)
      + lax.dot_general(bsum, tw[:, 0:8], dnT, preferred_element_type=f32)
      + lax.dot_general(csum, tw[:, 8:16], dnT, preferred_element_type=f32)
      + jnp.float32(_L) * tb_ref[...])
  a = pa_ref[0, 0]
  phrase = jnp.where(summed >= 0, summed, a * summed)

  oh = (lax.broadcasted_iota(jnp.int32, (_BLK, 128), 1) == lang_ref[...]).astype(f32)
  dn = (((1,), (0,)), ((), ()))
  lemb = lax.dot_general(oh, rl, dn, preferred_element_type=f32)

  dw = dw_ref[...]
  out_ref[...] = (
      lax.dot_general(lemb, dw[:, 0:16], dnT, preferred_element_type=f32)
      + lax.dot_general(phrase, dw[:, 16:], dnT, preferred_element_type=f32)
      + db_ref[...])


def _head(lang2, brk, caps, tsum, lt, bt, ct, tw, tb2, pa2, dw, db2,
          interpret=False):
  def ix(i):
    return (i, 0)

  def fx(i):
    return (0, 0)

  return pl.pallas_call(
      _head_body,
      grid=(_B // _BLK,),
      in_specs=[
          pl.BlockSpec((_BLK, 1), ix),
          pl.BlockSpec((_BLK, _L), ix),
          pl.BlockSpec((_BLK, _L), ix),
          pl.BlockSpec((_BLK, _TD), ix),
          pl.BlockSpec((128, 16), fx),
          pl.BlockSpec((4, 8), fx),
          pl.BlockSpec((4, 8), fx),
          pl.BlockSpec((64, 80), fx),
          pl.BlockSpec((1, 64), fx),
          pl.BlockSpec(memory_space=pltpu.SMEM),
          pl.BlockSpec((128, 80), fx),
          pl.BlockSpec((1, 128), fx),
      ],
      out_specs=pl.BlockSpec((_BLK, 128), ix),
      out_shape=jax.ShapeDtypeStruct((_B, 128), jnp.float32),
      interpret=interpret,
  )(lang2, brk, caps, tsum, lt, bt, ct, tw, tb2, pa2, dw, db2)


def kernel(lang_inputs, toks_inputs, lang_table, break_table, caps_table,
           text_table, toks_W, toks_b, prelu_a, docs_W, docs_b):
  brk = toks_inputs[0]
  caps = toks_inputs[1]
  tidx = toks_inputs[2].astype(jnp.int32)
  fmt = _format_table(text_table.T)
  b = tidx & 2047
  sidx = (tidx - b) + ((b & 1023) << 1) + (b >> 10)
  tsum = _text_bag(fmt.reshape(1000000, 64), sidx)

  pa2 = jnp.reshape(prelu_a, (1, 1))
  tb2 = jnp.reshape(toks_b, (1, 64))
  db2 = jnp.reshape(docs_b, (1, 128))
  return _head(lang_inputs, brk, caps, tsum, lang_table, break_table,
               caps_table, toks_W, tb2, pa2, docs_W, db2)
